# Initial kernel scaffold; baseline (speedup 1.0000x reference)
import jax
import jax.numpy as jnp
from jax.experimental import pallas as pl


def kernel(X, edge_index, H0_0, H0_1, H1_0, H1_1, H2_0, H2_1):
    raise NotImplementedError("write your pallas kernel here")


# speedup vs baseline: 33.1952x; 33.1952x over previous
import functools

import jax
import jax.numpy as jnp
from jax import lax
from jax.experimental import pallas as pl
from jax.experimental.pallas import tpu as pltpu
from jax.experimental.pallas import tpu_sc as plsc

_N = 100000
_E = 1600000
_NP = 100352
_EP = 1605632
_NB = _EP // 128
_BT32 = _NB // 32
_BT16 = _NB // 16
_K = 8
_NPT = _NP // 16
_ZRP = 196
_BN = 2048

_mesh = plsc.VectorSubcoreMesh(core_axis_name="c", subcore_axis_name="s")


def _elu(x):
    return jnp.where(x > 0.0, x, jnp.exp(jnp.minimum(x, 0.0)) - 1.0)


def _zero_acc(zbuf, acc, s):
    for z in range(_NPT // _ZRP):
        pltpu.sync_copy(zbuf, acc.at[pl.ds(s * _NPT + z * _ZRP, _ZRP)])


def _writeout(acc, wbuf, out, p, s):
    for z in range(_NPT // _ZRP):
        off = s * _NPT + z * _ZRP
        pltpu.sync_copy(acc.at[pl.ds(off, _ZRP)], wbuf)
        pltpu.sync_copy(wbuf, out.at[p, pl.ds(off, _ZRP), :])


@functools.partial(
    pl.kernel,
    out_type=jax.ShapeDtypeStruct((2, _NP, 16), jnp.float32),
    mesh=_mesh,
    compiler_params=pltpu.CompilerParams(use_tc_tiling_on_sc=False),
    scratch_types=[
        pltpu.VMEM((_K, 128), jnp.int32),
        pltpu.VMEM((128, 16), jnp.float32),
        pltpu.VMEM((_ZRP, 16), jnp.float32),
        pltpu.VMEM((_ZRP, 16), jnp.float32),
        pltpu.VMEM_SHARED((_NP, 16), jnp.float32),
        pltpu.SemaphoreType.DMA,
    ],
)
def _deg_kernel(coli, ones_hbm, zeros_hbm, out, sidx, ones_v, zbuf, wbuf,
                acc, sem):
    c = lax.axis_index("c")
    s = lax.axis_index("s")
    w = c * 16 + s
    pltpu.sync_copy(ones_hbm, ones_v)
    pltpu.sync_copy(zeros_hbm, zbuf)
    _zero_acc(zbuf, acc, s)
    plsc.subcore_barrier()

    def body(i, carry):
        base = w * _BT32 + i * _K
        pltpu.sync_copy(coli.at[pl.ds(base, _K)], sidx)
        descs = [
            pltpu.async_copy(ones_v, acc.at[sidx.at[j]], sem, add=True)
            for j in range(_K)
        ]
        for d in descs:
            d.wait()
        return carry

    lax.fori_loop(0, _BT32 // _K, body, 0)
    plsc.subcore_barrier()
    _writeout(acc, wbuf, out, c, s)


@functools.partial(
    pl.kernel,
    out_type=jax.ShapeDtypeStruct((2, _NP, 16), jnp.float32),
    mesh=_mesh,
    compiler_params=pltpu.CompilerParams(use_tc_tiling_on_sc=False),
    scratch_types=[
        pltpu.VMEM((_K, 128), jnp.int32),
        pltpu.VMEM((_K, 128), jnp.int32),
        pltpu.VMEM((_K * 128, 16), jnp.float32),
        pltpu.VMEM((_ZRP, 16), jnp.float32),
        pltpu.VMEM((_ZRP, 16), jnp.float32),
        pltpu.VMEM_SHARED((_NP, 16), jnp.float32),
        pltpu.SemaphoreType.DMA,
        pltpu.SemaphoreType.DMA,
    ],
)
def _prop16_kernel(tbl, rowi, coli, zeros_hbm, out,
                   gidx, sidx, rows, zbuf, wbuf, acc, semg, sems):
    c = lax.axis_index("c")
    s = lax.axis_index("s")
    w = c * 16 + s
    pltpu.sync_copy(zeros_hbm, zbuf)
    _zero_acc(zbuf, acc, s)
    plsc.subcore_barrier()

    def body(i, carry):
        base = w * _BT32 + i * _K
        pltpu.sync_copy(rowi.at[pl.ds(base, _K)], gidx)
        pltpu.sync_copy(coli.at[pl.ds(base, _K)], sidx)
        gds = [
            pltpu.async_copy(tbl.at[gidx.at[j]],
                             rows.at[pl.ds(j * 128, 128)], semg)
            for j in range(_K)
        ]
        for d in gds:
            d.wait()
        sds = [
            pltpu.async_copy(rows.at[pl.ds(j * 128, 128)],
                             acc.at[sidx.at[j]], sems, add=True)
            for j in range(_K)
        ]
        for d in sds:
            d.wait()
        return carry

    lax.fori_loop(0, _BT32 // _K, body, 0)
    plsc.subcore_barrier()
    _writeout(acc, wbuf, out, c, s)


@functools.partial(
    pl.kernel,
    out_type=jax.ShapeDtypeStruct((8, _NP, 16), jnp.float32),
    mesh=_mesh,
    compiler_params=pltpu.CompilerParams(use_tc_tiling_on_sc=False),
    scratch_types=[
        pltpu.VMEM((_K, 128), jnp.int32),
        pltpu.VMEM((_K, 128), jnp.int32),
        pltpu.VMEM((_K * 128, 16), jnp.float32),
        pltpu.VMEM((_ZRP, 16), jnp.float32),
        pltpu.VMEM((_ZRP, 16), jnp.float32),
        pltpu.VMEM_SHARED((_NP, 16), jnp.float32),
        pltpu.SemaphoreType.DMA,
        pltpu.SemaphoreType.DMA,
    ],
)
def _prop128_kernel(tbl, rowi, coli, zeros_hbm, out,
                    gidx, sidx, rows, zbuf, wbuf, acc, semg, sems):
    c = lax.axis_index("c")
    s = lax.axis_index("s")
    pltpu.sync_copy(zeros_hbm, zbuf)
    for pstep in range(4):
        p = pstep * 2 + c
        _zero_acc(zbuf, acc, s)
        plsc.subcore_barrier()

        def body(i, carry):
            base = s * _BT16 + i * _K
            pltpu.sync_copy(rowi.at[pl.ds(base, _K)], gidx)
            pltpu.sync_copy(coli.at[pl.ds(base, _K)], sidx)

            def tbody(j, c2):
                for l in range(8):
                    v = gidx[j, pl.ds(l * 16, 16)]
                    gidx[j, pl.ds(l * 16, 16)] = v * 8 + p
                return c2

            lax.fori_loop(0, _K, tbody, 0)
            gds = [
                pltpu.async_copy(tbl.at[gidx.at[j]],
                                 rows.at[pl.ds(j * 128, 128)], semg)
                for j in range(_K)
            ]
            for d in gds:
                d.wait()
            sds = [
                pltpu.async_copy(rows.at[pl.ds(j * 128, 128)],
                                 acc.at[sidx.at[j]], sems, add=True)
                for j in range(_K)
            ]
            for d in sds:
                d.wait()
            return carry

        lax.fori_loop(0, _BT16 // _K, body, 0)
        plsc.subcore_barrier()
        _writeout(acc, wbuf, out, p, s)
        plsc.subcore_barrier()


def _tca_body(deg_ref, x_ref, dis_ref, xs_ref):
    deg = deg_ref[0, :, 0:1] + deg_ref[1, :, 0:1]
    dis = jnp.where(deg > 0.0, lax.rsqrt(jnp.maximum(deg, 1e-12)), 0.0)
    dis_ref[...] = dis
    xs = dis * x_ref[...]
    xs_ref[...] = jnp.concatenate(
        [xs, jnp.zeros((xs.shape[0], 12), jnp.float32)], axis=1)


_tca = pl.pallas_call(
    _tca_body,
    grid=(_NP // _BN,),
    in_specs=[
        pl.BlockSpec((2, _BN, 16), lambda i: (0, i, 0)),
        pl.BlockSpec((_BN, 4), lambda i: (i, 0)),
    ],
    out_specs=[
        pl.BlockSpec((_BN, 1), lambda i: (i, 0)),
        pl.BlockSpec((_BN, 16), lambda i: (i, 0)),
    ],
    out_shape=[
        jax.ShapeDtypeStruct((_NP, 1), jnp.float32),
        jax.ShapeDtypeStruct((_NP, 16), jnp.float32),
    ],
)


def _tcb_body(x_ref, a0_ref, dis_ref, h00_ref, h01_ref, ys_ref):
    dis = dis_ref[...]
    p0 = dis * (a0_ref[0, :, 0:4] + a0_ref[1, :, 0:4])
    x = x_ref[...]
    h00 = h00_ref[...]
    h01 = h01_ref[...]
    cols = []
    for i in range(4):
        cols.append(_elu(x[:, i:i + 1] * h00 + p0[:, i:i + 1] * h01))
    y1 = jnp.concatenate(cols, axis=1)
    ys_ref[...] = dis * y1


_tcb = pl.pallas_call(
    _tcb_body,
    grid=(_NP // _BN,),
    in_specs=[
        pl.BlockSpec((_BN, 4), lambda i: (i, 0)),
        pl.BlockSpec((2, _BN, 16), lambda i: (0, i, 0)),
        pl.BlockSpec((_BN, 1), lambda i: (i, 0)),
        pl.BlockSpec((1, 32), lambda i: (0, 0)),
        pl.BlockSpec((1, 32), lambda i: (0, 0)),
    ],
    out_specs=pl.BlockSpec((_BN, 128), lambda i: (i, 0)),
    out_shape=jax.ShapeDtypeStruct((_NP, 128), jnp.float32),
)


def _tcc_body(x_ref, a0_ref, dis_ref, a1_ref,
              h00_ref, h01_ref, h10_ref, h11_ref, h20_ref, h21_ref,
              w_ref, zs_ref):
    dis = dis_ref[...]
    p0 = dis * (a0_ref[0, :, 0:4] + a0_ref[1, :, 0:4])
    x = x_ref[...]
    a1 = a1_ref[...]
    h00 = h00_ref[...]
    h01 = h01_ref[...]
    h10 = h10_ref[...]
    h11 = h11_ref[...]
    h20 = h20_ref[...]
    h21 = h21_ref[...]
    wcols = []
    zcols = []
    for i in range(4):
        y1 = _elu(x[:, i:i + 1] * h00 + p0[:, i:i + 1] * h01)
        p1 = dis * jnp.concatenate([a1[2 * i], a1[2 * i + 1]], axis=1)
        y2 = _elu(
            jnp.dot(y1, h10, preferred_element_type=jnp.float32)
            + jnp.dot(p1, h11, preferred_element_type=jnp.float32))
        wcols.append(jnp.dot(y2, h20, preferred_element_type=jnp.float32))
        zcols.append(jnp.dot(y2, h21, preferred_element_type=jnp.float32))
    w_ref[...] = jnp.concatenate(wcols, axis=1)
    zs = dis * jnp.concatenate(zcols, axis=1)
    zs_ref[...] = jnp.concatenate(
        [zs, jnp.zeros((zs.shape[0], 12), jnp.float32)], axis=1)


_tcc = pl.pallas_call(
    _tcc_body,
    grid=(_NP // _BN,),
    compiler_params=pltpu.CompilerParams(vmem_limit_bytes=100 * 1024 * 1024),
    in_specs=[
        pl.BlockSpec((_BN, 4), lambda i: (i, 0)),
        pl.BlockSpec((2, _BN, 16), lambda i: (0, i, 0)),
        pl.BlockSpec((_BN, 1), lambda i: (i, 0)),
        pl.BlockSpec((8, _BN, 16), lambda i: (0, i, 0)),
        pl.BlockSpec((1, 32), lambda i: (0, 0)),
        pl.BlockSpec((1, 32), lambda i: (0, 0)),
        pl.BlockSpec((32, 32), lambda i: (0, 0)),
        pl.BlockSpec((32, 32), lambda i: (0, 0)),
        pl.BlockSpec((32, 1), lambda i: (0, 0)),
        pl.BlockSpec((32, 1), lambda i: (0, 0)),
    ],
    out_specs=[
        pl.BlockSpec((_BN, 4), lambda i: (i, 0)),
        pl.BlockSpec((_BN, 16), lambda i: (i, 0)),
    ],
    out_shape=[
        jax.ShapeDtypeStruct((_NP, 4), jnp.float32),
        jax.ShapeDtypeStruct((_NP, 16), jnp.float32),
    ],
)


def _tcd_body(w_ref, dis_ref, a2_ref, o_ref):
    o_ref[...] = w_ref[...] + dis_ref[...] * (
        a2_ref[0, :, 0:4] + a2_ref[1, :, 0:4])


_tcd = pl.pallas_call(
    _tcd_body,
    grid=(_NP // _BN,),
    in_specs=[
        pl.BlockSpec((_BN, 4), lambda i: (i, 0)),
        pl.BlockSpec((_BN, 1), lambda i: (i, 0)),
        pl.BlockSpec((2, _BN, 16), lambda i: (0, i, 0)),
    ],
    out_specs=pl.BlockSpec((_BN, 4), lambda i: (i, 0)),
    out_shape=jax.ShapeDtypeStruct((_NP, 4), jnp.float32),
)


@jax.jit
def _stage1(X, edge_index):
    row = edge_index[0].astype(jnp.int32)
    col = edge_index[1].astype(jnp.int32)
    rowp = jnp.concatenate(
        [row, jnp.zeros((_EP - _E,), jnp.int32)]).reshape(_NB, 128)
    colp = jnp.concatenate(
        [col, jnp.full((_EP - _E,), _N, jnp.int32)]).reshape(_NB, 128)
    Xp = jnp.pad(X, ((0, _NP - _N), (0, 0)))
    deg2 = _deg_kernel(colp, jnp.ones((128, 16), jnp.float32),
                       jnp.zeros((_ZRP, 16), jnp.float32))
    dis, xs = _tca(deg2, Xp)
    return rowp, colp, Xp, dis, xs


@jax.jit
def _stage2(rowp, colp, Xp, dis, xs, H0_0, H0_1):
    acc0 = _prop16_kernel(xs, rowp, colp, jnp.zeros((_ZRP, 16), jnp.float32))
    ys1 = _tcb(Xp, acc0, dis, H0_0, H0_1)
    return acc0, ys1


@jax.jit
def _stage3(rowp, colp, Xp, dis, acc0, ys1,
            H0_0, H0_1, H1_0, H1_1, H2_0, H2_1):
    acc1 = _prop128_kernel(ys1.reshape(_NP * 8, 16), rowp, colp,
                           jnp.zeros((_ZRP, 16), jnp.float32))
    w4, zs = _tcc(Xp, acc0, dis, acc1, H0_0, H0_1, H1_0, H1_1, H2_0, H2_1)
    return w4, zs


@jax.jit
def _stage4(rowp, colp, dis, w4, zs):
    acc2 = _prop16_kernel(zs, rowp, colp, jnp.zeros((_ZRP, 16), jnp.float32))
    out = _tcd(w4, dis, acc2)
    return out[:_N]


def kernel(X, edge_index, H0_0, H0_1, H1_0, H1_1, H2_0, H2_1):
    rowp, colp, Xp, dis, xs = _stage1(X, edge_index)
    acc0, ys1 = _stage2(rowp, colp, Xp, dis, xs, H0_0, H0_1)
    w4, zs = _stage3(rowp, colp, Xp, dis, acc0, ys1,
                     H0_0, H0_1, H1_0, H1_1, H2_0, H2_1)
    return _stage4(rowp, colp, dis, w4, zs)


# speedup vs baseline: 35.4865x; 1.0690x over previous
import functools

import jax
import jax.numpy as jnp
from jax import lax
from jax.experimental import pallas as pl
from jax.experimental.pallas import tpu as pltpu
from jax.experimental.pallas import tpu_sc as plsc

_N = 100000
_E = 1600000
_NP = 100352
_EP = 1605632
_NB = _EP // 128
_BT32 = _NB // 32
_BT16 = _NB // 16
_K = 8
_NPT = _NP // 16
_ZRP = 196
_BN = 2048

_mesh = plsc.VectorSubcoreMesh(core_axis_name="c", subcore_axis_name="s")


def _elu(x):
    return jnp.where(x > 0.0, x, jnp.exp(jnp.minimum(x, 0.0)) - 1.0)


def _zero_acc(zbuf, acc, s):
    for z in range(_NPT // _ZRP):
        pltpu.sync_copy(zbuf, acc.at[pl.ds(s * _NPT + z * _ZRP, _ZRP)])


def _writeout(acc, wbuf, out, p, s):
    for z in range(_NPT // _ZRP):
        off = s * _NPT + z * _ZRP
        pltpu.sync_copy(acc.at[pl.ds(off, _ZRP)], wbuf)
        pltpu.sync_copy(wbuf, out.at[p, pl.ds(off, _ZRP), :])


@functools.partial(
    pl.kernel,
    out_type=jax.ShapeDtypeStruct((2, _NP, 16), jnp.float32),
    mesh=_mesh,
    compiler_params=pltpu.CompilerParams(use_tc_tiling_on_sc=False),
    scratch_types=[
        pltpu.VMEM((_K, 128), jnp.int32),
        pltpu.VMEM((128, 16), jnp.float32),
        pltpu.VMEM((_ZRP, 16), jnp.float32),
        pltpu.VMEM((_ZRP, 16), jnp.float32),
        pltpu.VMEM_SHARED((_NP, 16), jnp.float32),
        pltpu.SemaphoreType.DMA,
    ],
)
def _deg_kernel(coli, ones_hbm, zeros_hbm, out, sidx, ones_v, zbuf, wbuf,
                acc, sem):
    c = lax.axis_index("c")
    s = lax.axis_index("s")
    w = c * 16 + s
    pltpu.sync_copy(ones_hbm, ones_v)
    pltpu.sync_copy(zeros_hbm, zbuf)
    _zero_acc(zbuf, acc, s)
    plsc.subcore_barrier()

    def body(i, carry):
        base = w * _BT32 + i * _K
        pltpu.sync_copy(coli.at[pl.ds(base, _K)], sidx)
        descs = [
            pltpu.async_copy(ones_v, acc.at[sidx.at[j]], sem, add=True)
            for j in range(_K)
        ]
        for d in descs:
            d.wait()
        return carry

    lax.fori_loop(0, _BT32 // _K, body, 0)
    plsc.subcore_barrier()
    _writeout(acc, wbuf, out, c, s)


@functools.partial(
    pl.kernel,
    out_type=jax.ShapeDtypeStruct((2, _NP, 16), jnp.float32),
    mesh=_mesh,
    compiler_params=pltpu.CompilerParams(use_tc_tiling_on_sc=False),
    scratch_types=[
        pltpu.VMEM((_K, 128), jnp.int32),
        pltpu.VMEM((_K, 128), jnp.int32),
        pltpu.VMEM((_K * 128, 16), jnp.float32),
        pltpu.VMEM((_ZRP, 16), jnp.float32),
        pltpu.VMEM((_ZRP, 16), jnp.float32),
        pltpu.VMEM_SHARED((_NP, 16), jnp.float32),
        pltpu.SemaphoreType.DMA,
        pltpu.SemaphoreType.DMA,
    ],
)
def _prop16_kernel(tbl, rowi, coli, zeros_hbm, out,
                   gidx, sidx, rows, zbuf, wbuf, acc, semg, sems):
    c = lax.axis_index("c")
    s = lax.axis_index("s")
    w = c * 16 + s
    pltpu.sync_copy(zeros_hbm, zbuf)
    _zero_acc(zbuf, acc, s)
    plsc.subcore_barrier()

    def body(i, carry):
        base = w * _BT32 + i * _K
        pltpu.sync_copy(rowi.at[pl.ds(base, _K)], gidx)
        pltpu.sync_copy(coli.at[pl.ds(base, _K)], sidx)
        gds = [
            pltpu.async_copy(tbl.at[gidx.at[j]],
                             rows.at[pl.ds(j * 128, 128)], semg)
            for j in range(_K)
        ]
        sds = []
        for j in range(_K):
            gds[j].wait()
            sds.append(pltpu.async_copy(rows.at[pl.ds(j * 128, 128)],
                                        acc.at[sidx.at[j]], sems, add=True))
        for d in sds:
            d.wait()
        return carry

    lax.fori_loop(0, _BT32 // _K, body, 0)
    plsc.subcore_barrier()
    _writeout(acc, wbuf, out, c, s)


@functools.partial(
    pl.kernel,
    out_type=jax.ShapeDtypeStruct((8, _NP, 16), jnp.float32),
    mesh=_mesh,
    compiler_params=pltpu.CompilerParams(use_tc_tiling_on_sc=False),
    scratch_types=[
        pltpu.VMEM((_K, 128), jnp.int32),
        pltpu.VMEM((_K, 128), jnp.int32),
        pltpu.VMEM((_K * 128, 16), jnp.float32),
        pltpu.VMEM((_ZRP, 16), jnp.float32),
        pltpu.VMEM((_ZRP, 16), jnp.float32),
        pltpu.VMEM_SHARED((_NP, 16), jnp.float32),
        pltpu.SemaphoreType.DMA,
        pltpu.SemaphoreType.DMA,
    ],
)
def _prop128_kernel(tbl, rowi, coli, zeros_hbm, out,
                    gidx, sidx, rows, zbuf, wbuf, acc, semg, sems):
    c = lax.axis_index("c")
    s = lax.axis_index("s")
    pltpu.sync_copy(zeros_hbm, zbuf)
    for pstep in range(4):
        p = pstep * 2 + c
        _zero_acc(zbuf, acc, s)
        plsc.subcore_barrier()

        def body(i, carry):
            base = s * _BT16 + i * _K
            pltpu.sync_copy(rowi.at[pl.ds(base, _K)], gidx)
            pltpu.sync_copy(coli.at[pl.ds(base, _K)], sidx)

            def tbody(j, c2):
                for l in range(8):
                    v = gidx[j, pl.ds(l * 16, 16)]
                    gidx[j, pl.ds(l * 16, 16)] = v * 8 + p
                return c2

            lax.fori_loop(0, _K, tbody, 0)
            gds = [
                pltpu.async_copy(tbl.at[gidx.at[j]],
                                 rows.at[pl.ds(j * 128, 128)], semg)
                for j in range(_K)
            ]
            sds = []
            for j in range(_K):
                gds[j].wait()
                sds.append(pltpu.async_copy(
                    rows.at[pl.ds(j * 128, 128)],
                    acc.at[sidx.at[j]], sems, add=True))
            for d in sds:
                d.wait()
            return carry

        lax.fori_loop(0, _BT16 // _K, body, 0)
        plsc.subcore_barrier()
        _writeout(acc, wbuf, out, p, s)
        plsc.subcore_barrier()


def _tca_body(deg_ref, x_ref, dis_ref, xs_ref):
    deg = deg_ref[0, :, 0:1] + deg_ref[1, :, 0:1]
    dis = jnp.where(deg > 0.0, lax.rsqrt(jnp.maximum(deg, 1e-12)), 0.0)
    dis_ref[...] = dis
    xs = dis * x_ref[...]
    xs_ref[...] = jnp.concatenate(
        [xs, jnp.zeros((xs.shape[0], 12), jnp.float32)], axis=1)


_tca = pl.pallas_call(
    _tca_body,
    grid=(_NP // _BN,),
    in_specs=[
        pl.BlockSpec((2, _BN, 16), lambda i: (0, i, 0)),
        pl.BlockSpec((_BN, 4), lambda i: (i, 0)),
    ],
    out_specs=[
        pl.BlockSpec((_BN, 1), lambda i: (i, 0)),
        pl.BlockSpec((_BN, 16), lambda i: (i, 0)),
    ],
    out_shape=[
        jax.ShapeDtypeStruct((_NP, 1), jnp.float32),
        jax.ShapeDtypeStruct((_NP, 16), jnp.float32),
    ],
)


def _tcb_body(x_ref, a0_ref, dis_ref, h00_ref, h01_ref, ys_ref):
    dis = dis_ref[...]
    p0 = dis * (a0_ref[0, :, 0:4] + a0_ref[1, :, 0:4])
    x = x_ref[...]
    h00 = h00_ref[...]
    h01 = h01_ref[...]
    cols = []
    for i in range(4):
        cols.append(_elu(x[:, i:i + 1] * h00 + p0[:, i:i + 1] * h01))
    y1 = jnp.concatenate(cols, axis=1)
    ys_ref[...] = dis * y1


_tcb = pl.pallas_call(
    _tcb_body,
    grid=(_NP // _BN,),
    in_specs=[
        pl.BlockSpec((_BN, 4), lambda i: (i, 0)),
        pl.BlockSpec((2, _BN, 16), lambda i: (0, i, 0)),
        pl.BlockSpec((_BN, 1), lambda i: (i, 0)),
        pl.BlockSpec((1, 32), lambda i: (0, 0)),
        pl.BlockSpec((1, 32), lambda i: (0, 0)),
    ],
    out_specs=pl.BlockSpec((_BN, 128), lambda i: (i, 0)),
    out_shape=jax.ShapeDtypeStruct((_NP, 128), jnp.float32),
)


def _tcc_body(x_ref, a0_ref, dis_ref, a1_ref,
              h00_ref, h01_ref, h10_ref, h11_ref, h20_ref, h21_ref,
              w_ref, zs_ref):
    dis = dis_ref[...]
    p0 = dis * (a0_ref[0, :, 0:4] + a0_ref[1, :, 0:4])
    x = x_ref[...]
    a1 = a1_ref[...]
    h00 = h00_ref[...]
    h01 = h01_ref[...]
    h10 = h10_ref[...]
    h11 = h11_ref[...]
    h20 = h20_ref[...]
    h21 = h21_ref[...]
    wcols = []
    zcols = []
    for i in range(4):
        y1 = _elu(x[:, i:i + 1] * h00 + p0[:, i:i + 1] * h01)
        p1 = dis * jnp.concatenate([a1[2 * i], a1[2 * i + 1]], axis=1)
        y2 = _elu(
            jnp.dot(y1, h10, preferred_element_type=jnp.float32)
            + jnp.dot(p1, h11, preferred_element_type=jnp.float32))
        wcols.append(jnp.dot(y2, h20, preferred_element_type=jnp.float32))
        zcols.append(jnp.dot(y2, h21, preferred_element_type=jnp.float32))
    w_ref[...] = jnp.concatenate(wcols, axis=1)
    zs = dis * jnp.concatenate(zcols, axis=1)
    zs_ref[...] = jnp.concatenate(
        [zs, jnp.zeros((zs.shape[0], 12), jnp.float32)], axis=1)


_tcc = pl.pallas_call(
    _tcc_body,
    grid=(_NP // _BN,),
    compiler_params=pltpu.CompilerParams(vmem_limit_bytes=100 * 1024 * 1024),
    in_specs=[
        pl.BlockSpec((_BN, 4), lambda i: (i, 0)),
        pl.BlockSpec((2, _BN, 16), lambda i: (0, i, 0)),
        pl.BlockSpec((_BN, 1), lambda i: (i, 0)),
        pl.BlockSpec((8, _BN, 16), lambda i: (0, i, 0)),
        pl.BlockSpec((1, 32), lambda i: (0, 0)),
        pl.BlockSpec((1, 32), lambda i: (0, 0)),
        pl.BlockSpec((32, 32), lambda i: (0, 0)),
        pl.BlockSpec((32, 32), lambda i: (0, 0)),
        pl.BlockSpec((32, 1), lambda i: (0, 0)),
        pl.BlockSpec((32, 1), lambda i: (0, 0)),
    ],
    out_specs=[
        pl.BlockSpec((_BN, 4), lambda i: (i, 0)),
        pl.BlockSpec((_BN, 16), lambda i: (i, 0)),
    ],
    out_shape=[
        jax.ShapeDtypeStruct((_NP, 4), jnp.float32),
        jax.ShapeDtypeStruct((_NP, 16), jnp.float32),
    ],
)


def _tcd_body(w_ref, dis_ref, a2_ref, o_ref):
    o_ref[...] = w_ref[...] + dis_ref[...] * (
        a2_ref[0, :, 0:4] + a2_ref[1, :, 0:4])


_tcd = pl.pallas_call(
    _tcd_body,
    grid=(_NP // _BN,),
    in_specs=[
        pl.BlockSpec((_BN, 4), lambda i: (i, 0)),
        pl.BlockSpec((_BN, 1), lambda i: (i, 0)),
        pl.BlockSpec((2, _BN, 16), lambda i: (0, i, 0)),
    ],
    out_specs=pl.BlockSpec((_BN, 4), lambda i: (i, 0)),
    out_shape=jax.ShapeDtypeStruct((_NP, 4), jnp.float32),
)


@jax.jit
def _stage1(X, edge_index):
    row = edge_index[0].astype(jnp.int32)
    col = edge_index[1].astype(jnp.int32)
    rowp = jnp.concatenate(
        [row, jnp.zeros((_EP - _E,), jnp.int32)]).reshape(_NB, 128)
    colp = jnp.concatenate(
        [col, jnp.full((_EP - _E,), _N, jnp.int32)]).reshape(_NB, 128)
    Xp = jnp.pad(X, ((0, _NP - _N), (0, 0)))
    deg2 = _deg_kernel(colp, jnp.ones((128, 16), jnp.float32),
                       jnp.zeros((_ZRP, 16), jnp.float32))
    dis, xs = _tca(deg2, Xp)
    return rowp, colp, Xp, dis, xs


@jax.jit
def _stage2(rowp, colp, Xp, dis, xs, H0_0, H0_1):
    acc0 = _prop16_kernel(xs, rowp, colp, jnp.zeros((_ZRP, 16), jnp.float32))
    ys1 = _tcb(Xp, acc0, dis, H0_0, H0_1)
    return acc0, ys1


@jax.jit
def _stage3(rowp, colp, Xp, dis, acc0, ys1,
            H0_0, H0_1, H1_0, H1_1, H2_0, H2_1):
    acc1 = _prop128_kernel(ys1.reshape(_NP * 8, 16), rowp, colp,
                           jnp.zeros((_ZRP, 16), jnp.float32))
    w4, zs = _tcc(Xp, acc0, dis, acc1, H0_0, H0_1, H1_0, H1_1, H2_0, H2_1)
    return w4, zs


@jax.jit
def _stage4(rowp, colp, dis, w4, zs):
    acc2 = _prop16_kernel(zs, rowp, colp, jnp.zeros((_ZRP, 16), jnp.float32))
    out = _tcd(w4, dis, acc2)
    return out[:_N]


def kernel(X, edge_index, H0_0, H0_1, H1_0, H1_1, H2_0, H2_1):
    rowp, colp, Xp, dis, xs = _stage1(X, edge_index)
    acc0, ys1 = _stage2(rowp, colp, Xp, dis, xs, H0_0, H0_1)
    w4, zs = _stage3(rowp, colp, Xp, dis, acc0, ys1,
                     H0_0, H0_1, H1_0, H1_1, H2_0, H2_1)
    return _stage4(rowp, colp, dis, w4, zs)


# speedup vs baseline: 37.9767x; 1.0702x over previous
import functools

import jax
import jax.numpy as jnp
from jax import lax
from jax.experimental import pallas as pl
from jax.experimental.pallas import tpu as pltpu
from jax.experimental.pallas import tpu_sc as plsc

_N = 100000
_E = 1600000
_NP = 100352
_EP = 1605632
_NB = _EP // 128
_BT32 = _NB // 32
_BT16 = _NB // 16
_K = 8
_KP = 4
_NPT = _NP // 16
_ZRP = 196
_BN = 2048

_mesh = plsc.VectorSubcoreMesh(core_axis_name="c", subcore_axis_name="s")


def _elu(x):
    return jnp.where(x > 0.0, x, jnp.exp(jnp.minimum(x, 0.0)) - 1.0)


def _zero_acc(zbuf, acc, s):
    for z in range(_NPT // _ZRP):
        pltpu.sync_copy(zbuf, acc.at[pl.ds(s * _NPT + z * _ZRP, _ZRP)])


def _writeout(acc, wbuf, out, p, s):
    for z in range(_NPT // _ZRP):
        off = s * _NPT + z * _ZRP
        pltpu.sync_copy(acc.at[pl.ds(off, _ZRP)], wbuf)
        pltpu.sync_copy(wbuf, out.at[p, pl.ds(off, _ZRP), :])


@functools.partial(
    pl.kernel,
    out_type=jax.ShapeDtypeStruct((2, _NP, 16), jnp.float32),
    mesh=_mesh,
    compiler_params=pltpu.CompilerParams(use_tc_tiling_on_sc=False),
    scratch_types=[
        pltpu.VMEM((_K, 2, 128), jnp.int32),
        pltpu.VMEM((128, 16), jnp.float32),
        pltpu.VMEM((_ZRP, 16), jnp.float32),
        pltpu.VMEM((_ZRP, 16), jnp.float32),
        pltpu.VMEM_SHARED((_NP, 16), jnp.float32),
        pltpu.SemaphoreType.DMA,
    ],
)
def _deg_kernel(rc, ones_hbm, zeros_hbm, out, sidx, ones_v, zbuf, wbuf,
                acc, sem):
    c = lax.axis_index("c")
    s = lax.axis_index("s")
    w = c * 16 + s
    pltpu.sync_copy(ones_hbm, ones_v)
    pltpu.sync_copy(zeros_hbm, zbuf)
    _zero_acc(zbuf, acc, s)
    plsc.subcore_barrier()

    def body(i, carry):
        base = w * _BT32 + i * _K
        pltpu.sync_copy(rc.at[pl.ds(base, _K)], sidx)
        descs = [
            pltpu.async_copy(ones_v, acc.at[sidx.at[j, 1]], sem, add=True)
            for j in range(_K)
        ]
        for d in descs:
            d.wait()
        return carry

    lax.fori_loop(0, _BT32 // _K, body, 0)
    plsc.subcore_barrier()
    _writeout(acc, wbuf, out, c, s)


@functools.partial(
    pl.kernel,
    out_type=jax.ShapeDtypeStruct((2, _NP, 16), jnp.float32),
    mesh=_mesh,
    compiler_params=pltpu.CompilerParams(use_tc_tiling_on_sc=False),
    scratch_types=[
        pltpu.VMEM((_KP, 2, 128), jnp.int32),
        pltpu.VMEM((_KP, 2, 128), jnp.int32),
        pltpu.VMEM((_KP * 128, 16), jnp.float32),
        pltpu.VMEM((_KP * 128, 16), jnp.float32),
        pltpu.VMEM((_ZRP, 16), jnp.float32),
        pltpu.VMEM((_ZRP, 16), jnp.float32),
        pltpu.VMEM_SHARED((_NP, 16), jnp.float32),
        pltpu.SemaphoreType.DMA,
        pltpu.SemaphoreType.DMA,
    ],
)
def _prop16_kernel(tbl, rc, zeros_hbm, out,
                   rcA, rcB, rowsA, rowsB, zbuf, wbuf, acc, semg, sems):
    c = lax.axis_index("c")
    s = lax.axis_index("s")
    w = c * 16 + s
    pltpu.sync_copy(zeros_hbm, zbuf)
    _zero_acc(zbuf, acc, s)
    plsc.subcore_barrier()

    def _fire_gathers(rcb, rows, base):
        pltpu.sync_copy(rc.at[pl.ds(base, _KP)], rcb)
        return [
            pltpu.async_copy(tbl.at[rcb.at[j, 0]],
                             rows.at[pl.ds(j * 128, 128)], semg)
            for j in range(_KP)
        ]

    def body(i, carry):
        base = w * _BT32 + i * (2 * _KP)
        gA = _fire_gathers(rcA, rowsA, base)
        gB = _fire_gathers(rcB, rowsB, base + _KP)
        sds = []
        for rcb, rows, gds in ((rcA, rowsA, gA), (rcB, rowsB, gB)):
            for j in range(_KP):
                gds[j].wait()
                sds.append(pltpu.async_copy(
                    rows.at[pl.ds(j * 128, 128)],
                    acc.at[rcb.at[j, 1]], sems, add=True))
        for d in sds:
            d.wait()
        return carry

    lax.fori_loop(0, _BT32 // (2 * _KP), body, 0)
    plsc.subcore_barrier()
    _writeout(acc, wbuf, out, c, s)


@functools.partial(
    pl.kernel,
    out_type=jax.ShapeDtypeStruct((8, _NP, 16), jnp.float32),
    mesh=_mesh,
    compiler_params=pltpu.CompilerParams(use_tc_tiling_on_sc=False),
    scratch_types=[
        pltpu.VMEM((_KP, 2, 128), jnp.int32),
        pltpu.VMEM((_KP, 2, 128), jnp.int32),
        pltpu.VMEM((_KP * 128, 16), jnp.float32),
        pltpu.VMEM((_KP * 128, 16), jnp.float32),
        pltpu.VMEM((_ZRP, 16), jnp.float32),
        pltpu.VMEM((_ZRP, 16), jnp.float32),
        pltpu.VMEM_SHARED((_NP, 16), jnp.float32),
        pltpu.SemaphoreType.DMA,
        pltpu.SemaphoreType.DMA,
    ],
)
def _prop128_kernel(tbl, rc, zeros_hbm, out,
                    rcA, rcB, rowsA, rowsB, zbuf, wbuf, acc, semg, sems):
    c = lax.axis_index("c")
    s = lax.axis_index("s")
    pltpu.sync_copy(zeros_hbm, zbuf)
    for pstep in range(4):
        p = pstep * 2 + c
        _zero_acc(zbuf, acc, s)
        plsc.subcore_barrier()

        def _fire_gathers(rcb, rows, base):
            pltpu.sync_copy(rc.at[pl.ds(base, _KP)], rcb)
            for j in range(_KP):
                for l in range(8):
                    v = rcb[j, 0, pl.ds(l * 16, 16)]
                    rcb[j, 0, pl.ds(l * 16, 16)] = v * 8 + p
            return [
                pltpu.async_copy(tbl.at[rcb.at[j, 0]],
                                 rows.at[pl.ds(j * 128, 128)], semg)
                for j in range(_KP)
            ]

        def body(i, carry):
            base = s * _BT16 + i * (2 * _KP)
            gA = _fire_gathers(rcA, rowsA, base)
            gB = _fire_gathers(rcB, rowsB, base + _KP)
            sds = []
            for rcb, rows, gds in ((rcA, rowsA, gA), (rcB, rowsB, gB)):
                for j in range(_KP):
                    gds[j].wait()
                    sds.append(pltpu.async_copy(
                        rows.at[pl.ds(j * 128, 128)],
                        acc.at[rcb.at[j, 1]], sems, add=True))
            for d in sds:
                d.wait()
            return carry

        lax.fori_loop(0, _BT16 // (2 * _KP), body, 0)
        plsc.subcore_barrier()
        _writeout(acc, wbuf, out, p, s)
        plsc.subcore_barrier()


def _tca_body(deg_ref, x_ref, dis_ref, xs_ref):
    deg = deg_ref[0, :, 0:1] + deg_ref[1, :, 0:1]
    dis = jnp.where(deg > 0.0, lax.rsqrt(jnp.maximum(deg, 1e-12)), 0.0)
    dis_ref[...] = dis
    xs = dis * x_ref[...]
    xs_ref[...] = jnp.concatenate(
        [xs, jnp.zeros((xs.shape[0], 12), jnp.float32)], axis=1)


_tca = pl.pallas_call(
    _tca_body,
    grid=(_NP // _BN,),
    in_specs=[
        pl.BlockSpec((2, _BN, 16), lambda i: (0, i, 0)),
        pl.BlockSpec((_BN, 4), lambda i: (i, 0)),
    ],
    out_specs=[
        pl.BlockSpec((_BN, 1), lambda i: (i, 0)),
        pl.BlockSpec((_BN, 16), lambda i: (i, 0)),
    ],
    out_shape=[
        jax.ShapeDtypeStruct((_NP, 1), jnp.float32),
        jax.ShapeDtypeStruct((_NP, 16), jnp.float32),
    ],
)


def _tcb_body(x_ref, a0_ref, dis_ref, h00_ref, h01_ref, ys_ref):
    dis = dis_ref[...]
    p0 = dis * (a0_ref[0, :, 0:4] + a0_ref[1, :, 0:4])
    x = x_ref[...]
    h00 = h00_ref[...]
    h01 = h01_ref[...]
    cols = []
    for i in range(4):
        cols.append(_elu(x[:, i:i + 1] * h00 + p0[:, i:i + 1] * h01))
    y1 = jnp.concatenate(cols, axis=1)
    ys_ref[...] = dis * y1


_tcb = pl.pallas_call(
    _tcb_body,
    grid=(_NP // _BN,),
    in_specs=[
        pl.BlockSpec((_BN, 4), lambda i: (i, 0)),
        pl.BlockSpec((2, _BN, 16), lambda i: (0, i, 0)),
        pl.BlockSpec((_BN, 1), lambda i: (i, 0)),
        pl.BlockSpec((1, 32), lambda i: (0, 0)),
        pl.BlockSpec((1, 32), lambda i: (0, 0)),
    ],
    out_specs=pl.BlockSpec((_BN, 128), lambda i: (i, 0)),
    out_shape=jax.ShapeDtypeStruct((_NP, 128), jnp.float32),
)


def _tcc_body(x_ref, a0_ref, dis_ref, a1_ref,
              h00_ref, h01_ref, h10_ref, h11_ref, h20_ref, h21_ref,
              w_ref, zs_ref):
    dis = dis_ref[...]
    p0 = dis * (a0_ref[0, :, 0:4] + a0_ref[1, :, 0:4])
    x = x_ref[...]
    a1 = a1_ref[...]
    h00 = h00_ref[...]
    h01 = h01_ref[...]
    h10 = h10_ref[...]
    h11 = h11_ref[...]
    h20 = h20_ref[...]
    h21 = h21_ref[...]
    wcols = []
    zcols = []
    for i in range(4):
        y1 = _elu(x[:, i:i + 1] * h00 + p0[:, i:i + 1] * h01)
        p1 = dis * jnp.concatenate([a1[2 * i], a1[2 * i + 1]], axis=1)
        y2 = _elu(
            jnp.dot(y1, h10, preferred_element_type=jnp.float32)
            + jnp.dot(p1, h11, preferred_element_type=jnp.float32))
        wcols.append(jnp.dot(y2, h20, preferred_element_type=jnp.float32))
        zcols.append(jnp.dot(y2, h21, preferred_element_type=jnp.float32))
    w_ref[...] = jnp.concatenate(wcols, axis=1)
    zs = dis * jnp.concatenate(zcols, axis=1)
    zs_ref[...] = jnp.concatenate(
        [zs, jnp.zeros((zs.shape[0], 12), jnp.float32)], axis=1)


_tcc = pl.pallas_call(
    _tcc_body,
    grid=(_NP // _BN,),
    compiler_params=pltpu.CompilerParams(vmem_limit_bytes=100 * 1024 * 1024),
    in_specs=[
        pl.BlockSpec((_BN, 4), lambda i: (i, 0)),
        pl.BlockSpec((2, _BN, 16), lambda i: (0, i, 0)),
        pl.BlockSpec((_BN, 1), lambda i: (i, 0)),
        pl.BlockSpec((8, _BN, 16), lambda i: (0, i, 0)),
        pl.BlockSpec((1, 32), lambda i: (0, 0)),
        pl.BlockSpec((1, 32), lambda i: (0, 0)),
        pl.BlockSpec((32, 32), lambda i: (0, 0)),
        pl.BlockSpec((32, 32), lambda i: (0, 0)),
        pl.BlockSpec((32, 1), lambda i: (0, 0)),
        pl.BlockSpec((32, 1), lambda i: (0, 0)),
    ],
    out_specs=[
        pl.BlockSpec((_BN, 4), lambda i: (i, 0)),
        pl.BlockSpec((_BN, 16), lambda i: (i, 0)),
    ],
    out_shape=[
        jax.ShapeDtypeStruct((_NP, 4), jnp.float32),
        jax.ShapeDtypeStruct((_NP, 16), jnp.float32),
    ],
)


def _tcd_body(w_ref, dis_ref, a2_ref, o_ref):
    o_ref[...] = w_ref[...] + dis_ref[...] * (
        a2_ref[0, :, 0:4] + a2_ref[1, :, 0:4])


_tcd = pl.pallas_call(
    _tcd_body,
    grid=(_NP // _BN,),
    in_specs=[
        pl.BlockSpec((_BN, 4), lambda i: (i, 0)),
        pl.BlockSpec((_BN, 1), lambda i: (i, 0)),
        pl.BlockSpec((2, _BN, 16), lambda i: (0, i, 0)),
    ],
    out_specs=pl.BlockSpec((_BN, 4), lambda i: (i, 0)),
    out_shape=jax.ShapeDtypeStruct((_NP, 4), jnp.float32),
)


@jax.jit
def _stage1(X, edge_index):
    row = edge_index[0].astype(jnp.int32)
    col = edge_index[1].astype(jnp.int32)
    rowp = jnp.concatenate(
        [row, jnp.zeros((_EP - _E,), jnp.int32)]).reshape(_NB, 128)
    colp = jnp.concatenate(
        [col, jnp.full((_EP - _E,), _N, jnp.int32)]).reshape(_NB, 128)
    rc = jnp.stack([rowp, colp], axis=1)
    Xp = jnp.pad(X, ((0, _NP - _N), (0, 0)))
    deg2 = _deg_kernel(rc, jnp.ones((128, 16), jnp.float32),
                       jnp.zeros((_ZRP, 16), jnp.float32))
    dis, xs = _tca(deg2, Xp)
    return rc, Xp, dis, xs


@jax.jit
def _stage2(rc, Xp, dis, xs, H0_0, H0_1):
    acc0 = _prop16_kernel(xs, rc, jnp.zeros((_ZRP, 16), jnp.float32))
    ys1 = _tcb(Xp, acc0, dis, H0_0, H0_1)
    return acc0, ys1


@jax.jit
def _stage3(rc, Xp, dis, acc0, ys1,
            H0_0, H0_1, H1_0, H1_1, H2_0, H2_1):
    acc1 = _prop128_kernel(ys1.reshape(_NP * 8, 16), rc,
                           jnp.zeros((_ZRP, 16), jnp.float32))
    w4, zs = _tcc(Xp, acc0, dis, acc1, H0_0, H0_1, H1_0, H1_1, H2_0, H2_1)
    return w4, zs


@jax.jit
def _stage4(rc, dis, w4, zs):
    acc2 = _prop16_kernel(zs, rc, jnp.zeros((_ZRP, 16), jnp.float32))
    out = _tcd(w4, dis, acc2)
    return out[:_N]


def kernel(X, edge_index, H0_0, H0_1, H1_0, H1_1, H2_0, H2_1):
    rc, Xp, dis, xs = _stage1(X, edge_index)
    acc0, ys1 = _stage2(rc, Xp, dis, xs, H0_0, H0_1)
    w4, zs = _stage3(rc, Xp, dis, acc0, ys1,
                     H0_0, H0_1, H1_0, H1_1, H2_0, H2_1)
    return _stage4(rc, dis, w4, zs)


# speedup vs baseline: 40.6551x; 1.0705x over previous
import functools

import jax
import jax.numpy as jnp
from jax import lax
from jax.experimental import pallas as pl
from jax.experimental.pallas import tpu as pltpu
from jax.experimental.pallas import tpu_sc as plsc

_N = 100000
_E = 1600000
_NP = 100352
_EP = 1605632
_NB = _EP // 128
_BT32 = _NB // 32
_BT16 = _NB // 16
_K = 8
_KP = 4
_NPT = _NP // 16
_ZRP = 196
_BN = 2048

_mesh = plsc.VectorSubcoreMesh(core_axis_name="c", subcore_axis_name="s")


def _elu(x):
    return jnp.where(x > 0.0, x, jnp.exp(jnp.minimum(x, 0.0)) - 1.0)


def _zero_acc(zbuf, acc, s, sem):
    descs = [
        pltpu.async_copy(zbuf, acc.at[pl.ds(s * _NPT + z * _ZRP, _ZRP)], sem)
        for z in range(_NPT // _ZRP)
    ]
    for d in descs:
        d.wait()


def _writeout(acc, wbuf, out, p, s, sem):
    del wbuf
    descs = [
        pltpu.async_copy(acc.at[pl.ds(s * _NPT + z * _ZRP, _ZRP)],
                         out.at[p, pl.ds(s * _NPT + z * _ZRP, _ZRP), :], sem)
        for z in range(_NPT // _ZRP)
    ]
    for d in descs:
        d.wait()


@functools.partial(
    pl.kernel,
    out_type=jax.ShapeDtypeStruct((2, _NP, 16), jnp.float32),
    mesh=_mesh,
    compiler_params=pltpu.CompilerParams(use_tc_tiling_on_sc=False),
    scratch_types=[
        pltpu.VMEM((_K, 2, 128), jnp.int32),
        pltpu.VMEM((128, 16), jnp.float32),
        pltpu.VMEM((_ZRP, 16), jnp.float32),
        pltpu.VMEM((_ZRP, 16), jnp.float32),
        pltpu.VMEM_SHARED((_NP, 16), jnp.float32),
        pltpu.SemaphoreType.DMA,
    ],
)
def _deg_kernel(rc, ones_hbm, zeros_hbm, out, sidx, ones_v, zbuf, wbuf,
                acc, sem):
    c = lax.axis_index("c")
    s = lax.axis_index("s")
    w = c * 16 + s
    pltpu.sync_copy(ones_hbm, ones_v)
    pltpu.sync_copy(zeros_hbm, zbuf)
    _zero_acc(zbuf, acc, s, sem)
    plsc.subcore_barrier()

    def body(i, carry):
        base = w * _BT32 + i * _K
        pltpu.sync_copy(rc.at[pl.ds(base, _K)], sidx)
        descs = [
            pltpu.async_copy(ones_v, acc.at[sidx.at[j, 1]], sem, add=True)
            for j in range(_K)
        ]
        for d in descs:
            d.wait()
        return carry

    lax.fori_loop(0, _BT32 // _K, body, 0)
    plsc.subcore_barrier()
    _writeout(acc, wbuf, out, c, s, sem)


@functools.partial(
    pl.kernel,
    out_type=jax.ShapeDtypeStruct((2, _NP, 16), jnp.float32),
    mesh=_mesh,
    compiler_params=pltpu.CompilerParams(use_tc_tiling_on_sc=False),
    scratch_types=[
        pltpu.VMEM((_KP, 2, 128), jnp.int32),
        pltpu.VMEM((_KP, 2, 128), jnp.int32),
        pltpu.VMEM((_KP * 128, 16), jnp.float32),
        pltpu.VMEM((_KP * 128, 16), jnp.float32),
        pltpu.VMEM((_ZRP, 16), jnp.float32),
        pltpu.VMEM((_ZRP, 16), jnp.float32),
        pltpu.VMEM_SHARED((_NP, 16), jnp.float32),
        pltpu.SemaphoreType.DMA,
        pltpu.SemaphoreType.DMA,
    ],
)
def _prop16_kernel(tbl, rc, zeros_hbm, out,
                   rcA, rcB, rowsA, rowsB, zbuf, wbuf, acc, semg, sems):
    c = lax.axis_index("c")
    s = lax.axis_index("s")
    w = c * 16 + s
    pltpu.sync_copy(zeros_hbm, zbuf)
    _zero_acc(zbuf, acc, s, semg)
    plsc.subcore_barrier()

    def _fire_gathers(rcb, rows, base):
        pltpu.sync_copy(rc.at[pl.ds(base, _KP)], rcb)
        return [
            pltpu.async_copy(tbl.at[rcb.at[j, 0]],
                             rows.at[pl.ds(j * 128, 128)], semg)
            for j in range(_KP)
        ]

    def body(i, carry):
        base = w * _BT32 + i * (2 * _KP)
        gA = _fire_gathers(rcA, rowsA, base)
        gB = _fire_gathers(rcB, rowsB, base + _KP)
        sds = []
        for rcb, rows, gds in ((rcA, rowsA, gA), (rcB, rowsB, gB)):
            for j in range(_KP):
                gds[j].wait()
                sds.append(pltpu.async_copy(
                    rows.at[pl.ds(j * 128, 128)],
                    acc.at[rcb.at[j, 1]], sems, add=True))
        for d in sds:
            d.wait()
        return carry

    lax.fori_loop(0, _BT32 // (2 * _KP), body, 0)
    plsc.subcore_barrier()
    _writeout(acc, wbuf, out, c, s, semg)


@functools.partial(
    pl.kernel,
    out_type=jax.ShapeDtypeStruct((8, _NP, 16), jnp.float32),
    mesh=_mesh,
    compiler_params=pltpu.CompilerParams(use_tc_tiling_on_sc=False),
    scratch_types=[
        pltpu.VMEM((_KP, 2, 128), jnp.int32),
        pltpu.VMEM((_KP, 2, 128), jnp.int32),
        pltpu.VMEM((_KP * 128, 16), jnp.float32),
        pltpu.VMEM((_KP * 128, 16), jnp.float32),
        pltpu.VMEM((_ZRP, 16), jnp.float32),
        pltpu.VMEM((_ZRP, 16), jnp.float32),
        pltpu.VMEM_SHARED((_NP, 16), jnp.float32),
        pltpu.SemaphoreType.DMA,
        pltpu.SemaphoreType.DMA,
    ],
)
def _prop128_kernel(tbl, rc, zeros_hbm, out,
                    rcA, rcB, rowsA, rowsB, zbuf, wbuf, acc, semg, sems):
    c = lax.axis_index("c")
    s = lax.axis_index("s")
    pltpu.sync_copy(zeros_hbm, zbuf)
    for pstep in range(4):
        p = pstep * 2 + c
        _zero_acc(zbuf, acc, s, semg)
        plsc.subcore_barrier()

        def _fire_gathers(rcb, rows, base):
            pltpu.sync_copy(rc.at[pl.ds(base, _KP)], rcb)
            for j in range(_KP):
                for l in range(8):
                    v = rcb[j, 0, pl.ds(l * 16, 16)]
                    rcb[j, 0, pl.ds(l * 16, 16)] = v * 8 + p
            return [
                pltpu.async_copy(tbl.at[rcb.at[j, 0]],
                                 rows.at[pl.ds(j * 128, 128)], semg)
                for j in range(_KP)
            ]

        def body(i, carry):
            base = s * _BT16 + i * (2 * _KP)
            gA = _fire_gathers(rcA, rowsA, base)
            gB = _fire_gathers(rcB, rowsB, base + _KP)
            sds = []
            for rcb, rows, gds in ((rcA, rowsA, gA), (rcB, rowsB, gB)):
                for j in range(_KP):
                    gds[j].wait()
                    sds.append(pltpu.async_copy(
                        rows.at[pl.ds(j * 128, 128)],
                        acc.at[rcb.at[j, 1]], sems, add=True))
            for d in sds:
                d.wait()
            return carry

        lax.fori_loop(0, _BT16 // (2 * _KP), body, 0)
        plsc.subcore_barrier()
        _writeout(acc, wbuf, out, p, s, semg)
        plsc.subcore_barrier()


def _tca_body(deg_ref, x_ref, dis_ref, xs_ref):
    deg = deg_ref[0, :, 0:1] + deg_ref[1, :, 0:1]
    dis = jnp.where(deg > 0.0, lax.rsqrt(jnp.maximum(deg, 1e-12)), 0.0)
    dis_ref[...] = dis
    xs = dis * x_ref[...]
    xs_ref[...] = jnp.concatenate(
        [xs, jnp.zeros((xs.shape[0], 12), jnp.float32)], axis=1)


_tca = pl.pallas_call(
    _tca_body,
    grid=(_NP // _BN,),
    in_specs=[
        pl.BlockSpec((2, _BN, 16), lambda i: (0, i, 0)),
        pl.BlockSpec((_BN, 4), lambda i: (i, 0)),
    ],
    out_specs=[
        pl.BlockSpec((_BN, 1), lambda i: (i, 0)),
        pl.BlockSpec((_BN, 16), lambda i: (i, 0)),
    ],
    out_shape=[
        jax.ShapeDtypeStruct((_NP, 1), jnp.float32),
        jax.ShapeDtypeStruct((_NP, 16), jnp.float32),
    ],
)


def _tcb_body(x_ref, a0_ref, dis_ref, h00k_ref, h01k_ref, ys_ref):
    dis = dis_ref[...]
    p0 = dis * (a0_ref[0, :, 0:4] + a0_ref[1, :, 0:4])
    x = x_ref[...]
    y1 = _elu(jnp.dot(x, h00k_ref[...], preferred_element_type=jnp.float32)
              + jnp.dot(p0, h01k_ref[...], preferred_element_type=jnp.float32))
    ys_ref[...] = dis * y1


_tcb = pl.pallas_call(
    _tcb_body,
    grid=(_NP // _BN,),
    in_specs=[
        pl.BlockSpec((_BN, 4), lambda i: (i, 0)),
        pl.BlockSpec((2, _BN, 16), lambda i: (0, i, 0)),
        pl.BlockSpec((_BN, 1), lambda i: (i, 0)),
        pl.BlockSpec((4, 128), lambda i: (0, 0)),
        pl.BlockSpec((4, 128), lambda i: (0, 0)),
    ],
    out_specs=pl.BlockSpec((_BN, 128), lambda i: (i, 0)),
    out_shape=jax.ShapeDtypeStruct((_NP, 128), jnp.float32),
)


def _tcc_body(x_ref, a0_ref, dis_ref, a1_ref,
              h00k_ref, h01k_ref, h10k_ref, h11k_ref, h20k_ref, h21k_ref,
              w_ref, zs_ref):
    dis = dis_ref[...]
    p0 = dis * (a0_ref[0, :, 0:4] + a0_ref[1, :, 0:4])
    x = x_ref[...]
    a1 = a1_ref[...]
    y1 = _elu(jnp.dot(x, h00k_ref[...], preferred_element_type=jnp.float32)
              + jnp.dot(p0, h01k_ref[...], preferred_element_type=jnp.float32))
    p1 = dis * jnp.concatenate([a1[q] for q in range(8)], axis=1)
    y2 = _elu(
        jnp.dot(y1, h10k_ref[...], preferred_element_type=jnp.float32)
        + jnp.dot(p1, h11k_ref[...], preferred_element_type=jnp.float32))
    w_ref[...] = jnp.dot(y2, h20k_ref[...],
                         preferred_element_type=jnp.float32)
    zs = dis * jnp.dot(y2, h21k_ref[...], preferred_element_type=jnp.float32)
    zs_ref[...] = jnp.concatenate(
        [zs, jnp.zeros((zs.shape[0], 12), jnp.float32)], axis=1)


_tcc = pl.pallas_call(
    _tcc_body,
    grid=(_NP // _BN,),
    compiler_params=pltpu.CompilerParams(vmem_limit_bytes=100 * 1024 * 1024),
    in_specs=[
        pl.BlockSpec((_BN, 4), lambda i: (i, 0)),
        pl.BlockSpec((2, _BN, 16), lambda i: (0, i, 0)),
        pl.BlockSpec((_BN, 1), lambda i: (i, 0)),
        pl.BlockSpec((8, _BN, 16), lambda i: (0, i, 0)),
        pl.BlockSpec((4, 128), lambda i: (0, 0)),
        pl.BlockSpec((4, 128), lambda i: (0, 0)),
        pl.BlockSpec((128, 128), lambda i: (0, 0)),
        pl.BlockSpec((128, 128), lambda i: (0, 0)),
        pl.BlockSpec((128, 4), lambda i: (0, 0)),
        pl.BlockSpec((128, 4), lambda i: (0, 0)),
    ],
    out_specs=[
        pl.BlockSpec((_BN, 4), lambda i: (i, 0)),
        pl.BlockSpec((_BN, 16), lambda i: (i, 0)),
    ],
    out_shape=[
        jax.ShapeDtypeStruct((_NP, 4), jnp.float32),
        jax.ShapeDtypeStruct((_NP, 16), jnp.float32),
    ],
)


def _tcd_body(w_ref, dis_ref, a2_ref, o_ref):
    o_ref[...] = w_ref[...] + dis_ref[...] * (
        a2_ref[0, :, 0:4] + a2_ref[1, :, 0:4])


_tcd = pl.pallas_call(
    _tcd_body,
    grid=(_NP // _BN,),
    in_specs=[
        pl.BlockSpec((_BN, 4), lambda i: (i, 0)),
        pl.BlockSpec((_BN, 1), lambda i: (i, 0)),
        pl.BlockSpec((2, _BN, 16), lambda i: (0, i, 0)),
    ],
    out_specs=pl.BlockSpec((_BN, 4), lambda i: (i, 0)),
    out_shape=jax.ShapeDtypeStruct((_NP, 4), jnp.float32),
)


@jax.jit
def _stage1(X, edge_index):
    row = edge_index[0].astype(jnp.int32)
    col = edge_index[1].astype(jnp.int32)
    rowp = jnp.concatenate(
        [row, jnp.zeros((_EP - _E,), jnp.int32)]).reshape(_NB, 128)
    colp = jnp.concatenate(
        [col, jnp.full((_EP - _E,), _N, jnp.int32)]).reshape(_NB, 128)
    rc = jnp.stack([rowp, colp], axis=1)
    Xp = jnp.pad(X, ((0, _NP - _N), (0, 0)))
    deg2 = _deg_kernel(rc, jnp.ones((128, 16), jnp.float32),
                       jnp.zeros((_ZRP, 16), jnp.float32))
    dis, xs = _tca(deg2, Xp)
    return rc, Xp, dis, xs


def _kron4(h):
    return jnp.kron(jnp.eye(4, dtype=jnp.float32), h)


@jax.jit
def _stage2(rc, Xp, dis, xs, H0_0, H0_1):
    acc0 = _prop16_kernel(xs, rc, jnp.zeros((_ZRP, 16), jnp.float32))
    ys1 = _tcb(Xp, acc0, dis, _kron4(H0_0), _kron4(H0_1))
    return acc0, ys1


@jax.jit
def _stage3(rc, Xp, dis, acc0, ys1,
            H0_0, H0_1, H1_0, H1_1, H2_0, H2_1):
    acc1 = _prop128_kernel(ys1.reshape(_NP * 8, 16), rc,
                           jnp.zeros((_ZRP, 16), jnp.float32))
    w4, zs = _tcc(Xp, acc0, dis, acc1, _kron4(H0_0), _kron4(H0_1),
                  _kron4(H1_0), _kron4(H1_1), _kron4(H2_0), _kron4(H2_1))
    return w4, zs


@jax.jit
def _stage4(rc, dis, w4, zs):
    acc2 = _prop16_kernel(zs, rc, jnp.zeros((_ZRP, 16), jnp.float32))
    out = _tcd(w4, dis, acc2)
    return out[:_N]


def kernel(X, edge_index, H0_0, H0_1, H1_0, H1_1, H2_0, H2_1):
    rc, Xp, dis, xs = _stage1(X, edge_index)
    acc0, ys1 = _stage2(rc, Xp, dis, xs, H0_0, H0_1)
    w4, zs = _stage3(rc, Xp, dis, acc0, ys1,
                     H0_0, H0_1, H1_0, H1_1, H2_0, H2_1)
    return _stage4(rc, dis, w4, zs)


# speedup vs baseline: 41.4192x; 1.0188x over previous
import functools

import jax
import jax.numpy as jnp
from jax import lax
from jax.experimental import pallas as pl
from jax.experimental.pallas import tpu as pltpu
from jax.experimental.pallas import tpu_sc as plsc

_N = 100000
_E = 1600000
_NP = 100352
_EP = 1605632
_NB = _EP // 128
_BT32 = _NB // 32
_BT16 = _NB // 16
_K = 8
_KP = 4
_NPT = _NP // 16
_ZRP = 196
_BN = 2048

_mesh = plsc.VectorSubcoreMesh(core_axis_name="c", subcore_axis_name="s")


def _elu(x):
    return jnp.where(x > 0.0, x, jnp.exp(jnp.minimum(x, 0.0)) - 1.0)


def _zero_acc(zbuf, acc, s, sem):
    descs = [
        pltpu.async_copy(zbuf, acc.at[pl.ds(s * _NPT + z * _ZRP, _ZRP)], sem)
        for z in range(_NPT // _ZRP)
    ]
    for d in descs:
        d.wait()


def _writeout(acc, wbuf, out, p, s, sem):
    del wbuf
    descs = [
        pltpu.async_copy(acc.at[pl.ds(s * _NPT + z * _ZRP, _ZRP)],
                         out.at[p, pl.ds(s * _NPT + z * _ZRP, _ZRP), :], sem)
        for z in range(_NPT // _ZRP)
    ]
    for d in descs:
        d.wait()


@functools.partial(
    pl.kernel,
    out_type=jax.ShapeDtypeStruct((2, _NP, 16), jnp.float32),
    mesh=_mesh,
    compiler_params=pltpu.CompilerParams(use_tc_tiling_on_sc=False),
    scratch_types=[
        pltpu.VMEM((_K, 2, 128), jnp.int32),
        pltpu.VMEM((128, 16), jnp.float32),
        pltpu.VMEM((_ZRP, 16), jnp.float32),
        pltpu.VMEM((_ZRP, 16), jnp.float32),
        pltpu.VMEM_SHARED((_NP, 16), jnp.float32),
        pltpu.SemaphoreType.DMA,
    ],
)
def _deg_kernel(rc, ones_hbm, zeros_hbm, out, sidx, ones_v, zbuf, wbuf,
                acc, sem):
    c = lax.axis_index("c")
    s = lax.axis_index("s")
    w = c * 16 + s
    pltpu.sync_copy(ones_hbm, ones_v)
    pltpu.sync_copy(zeros_hbm, zbuf)
    _zero_acc(zbuf, acc, s, sem)
    plsc.subcore_barrier()

    def body(i, carry):
        base = w * _BT32 + i * _K
        pltpu.sync_copy(rc.at[pl.ds(base, _K)], sidx)
        descs = [
            pltpu.async_copy(ones_v, acc.at[sidx.at[j, 1]], sem, add=True)
            for j in range(_K)
        ]
        for d in descs:
            d.wait()
        return carry

    lax.fori_loop(0, _BT32 // _K, body, 0)
    plsc.subcore_barrier()
    _writeout(acc, wbuf, out, c, s, sem)


@functools.partial(
    pl.kernel,
    out_type=jax.ShapeDtypeStruct((2, _NP, 16), jnp.float32),
    mesh=_mesh,
    compiler_params=pltpu.CompilerParams(use_tc_tiling_on_sc=False),
    scratch_types=[
        pltpu.VMEM((_KP, 2, 128), jnp.int32),
        pltpu.VMEM((_KP, 2, 128), jnp.int32),
        pltpu.VMEM((_KP * 128, 16), jnp.float32),
        pltpu.VMEM((_KP * 128, 16), jnp.float32),
        pltpu.VMEM((_ZRP, 16), jnp.float32),
        pltpu.VMEM((_ZRP, 16), jnp.float32),
        pltpu.VMEM_SHARED((_NP, 16), jnp.float32),
        pltpu.SemaphoreType.DMA,
        pltpu.SemaphoreType.DMA,
    ],
)
def _prop16_kernel(tbl, rc, zeros_hbm, out,
                   rcA, rcB, rowsA, rowsB, zbuf, wbuf, acc, semg, sems):
    c = lax.axis_index("c")
    s = lax.axis_index("s")
    w = c * 16 + s
    pltpu.sync_copy(zeros_hbm, zbuf)
    _zero_acc(zbuf, acc, s, semg)
    plsc.subcore_barrier()

    def _fire_gathers(rcb, rows, base):
        pltpu.sync_copy(rc.at[pl.ds(base, _KP)], rcb)
        return [
            pltpu.async_copy(tbl.at[rcb.at[j, 0]],
                             rows.at[pl.ds(j * 128, 128)], semg)
            for j in range(_KP)
        ]

    def body(i, carry):
        base = w * _BT32 + i * (2 * _KP)
        gA = _fire_gathers(rcA, rowsA, base)
        gB = _fire_gathers(rcB, rowsB, base + _KP)
        sds = []
        for rcb, rows, gds in ((rcA, rowsA, gA), (rcB, rowsB, gB)):
            for j in range(_KP):
                gds[j].wait()
                sds.append(pltpu.async_copy(
                    rows.at[pl.ds(j * 128, 128)],
                    acc.at[rcb.at[j, 1]], sems, add=True))
        for d in sds:
            d.wait()
        return carry

    lax.fori_loop(0, _BT32 // (2 * _KP), body, 0)
    plsc.subcore_barrier()
    _writeout(acc, wbuf, out, c, s, semg)


@functools.partial(
    pl.kernel,
    out_type=jax.ShapeDtypeStruct((8, _NP, 16), jnp.float32),
    mesh=_mesh,
    compiler_params=pltpu.CompilerParams(use_tc_tiling_on_sc=False),
    scratch_types=[
        pltpu.VMEM((_KP, 2, 128), jnp.int32),
        pltpu.VMEM((_KP, 2, 128), jnp.int32),
        pltpu.VMEM((_KP * 128, 16), jnp.float32),
        pltpu.VMEM((_KP * 128, 16), jnp.float32),
        pltpu.VMEM((_ZRP, 16), jnp.float32),
        pltpu.VMEM((_ZRP, 16), jnp.float32),
        pltpu.VMEM_SHARED((_NP, 16), jnp.float32),
        pltpu.SemaphoreType.DMA,
        pltpu.SemaphoreType.DMA,
        pltpu.SemaphoreType.DMA,
    ],
)
def _prop128_kernel(tbl, rc, zeros_hbm, out,
                    rcA, rcB, rowsA, rowsB, zbuf, wbuf, acc,
                    semg, sems, semsb):
    c = lax.axis_index("c")
    s = lax.axis_index("s")
    pltpu.sync_copy(zeros_hbm, zbuf)
    for pstep in range(4):
        p = pstep * 2 + c
        _zero_acc(zbuf, acc, s, semg)
        plsc.subcore_barrier()

        def _fire_gathers(rcb, rows, base):
            pltpu.sync_copy(rc.at[pl.ds(base, _KP)], rcb)
            for j in range(_KP):
                for l in range(8):
                    v = rcb[j, 0, pl.ds(l * 16, 16)]
                    rcb[j, 0, pl.ds(l * 16, 16)] = v * 8 + p
            return [
                pltpu.async_copy(tbl.at[rcb.at[j, 0]],
                                 rows.at[pl.ds(j * 128, 128)], semg)
                for j in range(_KP)
            ]

        def _drain_b():
            for j in range(_KP):
                pltpu.make_async_copy(rowsB.at[pl.ds(j * 128, 128)],
                                      acc.at[rcB.at[j, 1]], semsb).wait()

        def body(i, carry):
            base = s * _BT16 + i * (2 * _KP)
            gA = _fire_gathers(rcA, rowsA, base)

            @pl.when(i > 0)
            def _():
                _drain_b()

            gB = _fire_gathers(rcB, rowsB, base + _KP)
            sds = []
            for j in range(_KP):
                gA[j].wait()
                sds.append(pltpu.async_copy(
                    rowsA.at[pl.ds(j * 128, 128)],
                    acc.at[rcA.at[j, 1]], sems, add=True))
            for j in range(_KP):
                gB[j].wait()
                pltpu.async_copy(rowsB.at[pl.ds(j * 128, 128)],
                                 acc.at[rcB.at[j, 1]], semsb, add=True)
            for d in sds:
                d.wait()
            return carry

        lax.fori_loop(0, _BT16 // (2 * _KP), body, 0)
        _drain_b()
        plsc.subcore_barrier()
        _writeout(acc, wbuf, out, p, s, semg)
        plsc.subcore_barrier()


def _tca_body(deg_ref, x_ref, dis_ref, xs_ref):
    deg = deg_ref[0, :, 0:1] + deg_ref[1, :, 0:1]
    dis = jnp.where(deg > 0.0, lax.rsqrt(jnp.maximum(deg, 1e-12)), 0.0)
    dis_ref[...] = dis
    xs = dis * x_ref[...]
    xs_ref[...] = jnp.concatenate(
        [xs, jnp.zeros((xs.shape[0], 12), jnp.float32)], axis=1)


_tca = pl.pallas_call(
    _tca_body,
    grid=(_NP // _BN,),
    in_specs=[
        pl.BlockSpec((2, _BN, 16), lambda i: (0, i, 0)),
        pl.BlockSpec((_BN, 4), lambda i: (i, 0)),
    ],
    out_specs=[
        pl.BlockSpec((_BN, 1), lambda i: (i, 0)),
        pl.BlockSpec((_BN, 16), lambda i: (i, 0)),
    ],
    out_shape=[
        jax.ShapeDtypeStruct((_NP, 1), jnp.float32),
        jax.ShapeDtypeStruct((_NP, 16), jnp.float32),
    ],
)


def _tcb_body(x_ref, a0_ref, dis_ref, h00k_ref, h01k_ref, ys_ref):
    dis = dis_ref[...]
    p0 = dis * (a0_ref[0, :, 0:4] + a0_ref[1, :, 0:4])
    x = x_ref[...]
    y1 = _elu(jnp.dot(x, h00k_ref[...], preferred_element_type=jnp.float32)
              + jnp.dot(p0, h01k_ref[...], preferred_element_type=jnp.float32))
    ys_ref[...] = dis * y1


_tcb = pl.pallas_call(
    _tcb_body,
    grid=(_NP // _BN,),
    in_specs=[
        pl.BlockSpec((_BN, 4), lambda i: (i, 0)),
        pl.BlockSpec((2, _BN, 16), lambda i: (0, i, 0)),
        pl.BlockSpec((_BN, 1), lambda i: (i, 0)),
        pl.BlockSpec((4, 128), lambda i: (0, 0)),
        pl.BlockSpec((4, 128), lambda i: (0, 0)),
    ],
    out_specs=pl.BlockSpec((_BN, 128), lambda i: (i, 0)),
    out_shape=jax.ShapeDtypeStruct((_NP, 128), jnp.float32),
)


def _tcc_body(x_ref, a0_ref, dis_ref, a1_ref,
              h00k_ref, h01k_ref, h10k_ref, h11k_ref, h20k_ref, h21k_ref,
              w_ref, zs_ref):
    dis = dis_ref[...]
    p0 = dis * (a0_ref[0, :, 0:4] + a0_ref[1, :, 0:4])
    x = x_ref[...]
    a1 = a1_ref[...]
    y1 = _elu(jnp.dot(x, h00k_ref[...], preferred_element_type=jnp.float32)
              + jnp.dot(p0, h01k_ref[...], preferred_element_type=jnp.float32))
    p1 = dis * jnp.concatenate([a1[q] for q in range(8)], axis=1)
    y2 = _elu(
        jnp.dot(y1, h10k_ref[...], preferred_element_type=jnp.float32)
        + jnp.dot(p1, h11k_ref[...], preferred_element_type=jnp.float32))
    w_ref[...] = jnp.dot(y2, h20k_ref[...],
                         preferred_element_type=jnp.float32)
    zs = dis * jnp.dot(y2, h21k_ref[...], preferred_element_type=jnp.float32)
    zs_ref[...] = jnp.concatenate(
        [zs, jnp.zeros((zs.shape[0], 12), jnp.float32)], axis=1)


_tcc = pl.pallas_call(
    _tcc_body,
    grid=(_NP // _BN,),
    compiler_params=pltpu.CompilerParams(vmem_limit_bytes=100 * 1024 * 1024),
    in_specs=[
        pl.BlockSpec((_BN, 4), lambda i: (i, 0)),
        pl.BlockSpec((2, _BN, 16), lambda i: (0, i, 0)),
        pl.BlockSpec((_BN, 1), lambda i: (i, 0)),
        pl.BlockSpec((8, _BN, 16), lambda i: (0, i, 0)),
        pl.BlockSpec((4, 128), lambda i: (0, 0)),
        pl.BlockSpec((4, 128), lambda i: (0, 0)),
        pl.BlockSpec((128, 128), lambda i: (0, 0)),
        pl.BlockSpec((128, 128), lambda i: (0, 0)),
        pl.BlockSpec((128, 4), lambda i: (0, 0)),
        pl.BlockSpec((128, 4), lambda i: (0, 0)),
    ],
    out_specs=[
        pl.BlockSpec((_BN, 4), lambda i: (i, 0)),
        pl.BlockSpec((_BN, 16), lambda i: (i, 0)),
    ],
    out_shape=[
        jax.ShapeDtypeStruct((_NP, 4), jnp.float32),
        jax.ShapeDtypeStruct((_NP, 16), jnp.float32),
    ],
)


def _tcd_body(w_ref, dis_ref, a2_ref, o_ref):
    o_ref[...] = w_ref[...] + dis_ref[...] * (
        a2_ref[0, :, 0:4] + a2_ref[1, :, 0:4])


_tcd = pl.pallas_call(
    _tcd_body,
    grid=(_NP // _BN,),
    in_specs=[
        pl.BlockSpec((_BN, 4), lambda i: (i, 0)),
        pl.BlockSpec((_BN, 1), lambda i: (i, 0)),
        pl.BlockSpec((2, _BN, 16), lambda i: (0, i, 0)),
    ],
    out_specs=pl.BlockSpec((_BN, 4), lambda i: (i, 0)),
    out_shape=jax.ShapeDtypeStruct((_NP, 4), jnp.float32),
)


@jax.jit
def _stage1(X, edge_index):
    row = edge_index[0].astype(jnp.int32)
    col = edge_index[1].astype(jnp.int32)
    rowp = jnp.concatenate(
        [row, jnp.zeros((_EP - _E,), jnp.int32)]).reshape(_NB, 128)
    colp = jnp.concatenate(
        [col, jnp.full((_EP - _E,), _N, jnp.int32)]).reshape(_NB, 128)
    rc = jnp.stack([rowp, colp], axis=1)
    Xp = jnp.pad(X, ((0, _NP - _N), (0, 0)))
    deg2 = _deg_kernel(rc, jnp.ones((128, 16), jnp.float32),
                       jnp.zeros((_ZRP, 16), jnp.float32))
    dis, xs = _tca(deg2, Xp)
    return rc, Xp, dis, xs


def _kron4(h):
    return jnp.kron(jnp.eye(4, dtype=jnp.float32), h)


@jax.jit
def _stage2(rc, Xp, dis, xs, H0_0, H0_1):
    acc0 = _prop16_kernel(xs, rc, jnp.zeros((_ZRP, 16), jnp.float32))
    ys1 = _tcb(Xp, acc0, dis, _kron4(H0_0), _kron4(H0_1))
    return acc0, ys1


@jax.jit
def _stage3(rc, Xp, dis, acc0, ys1,
            H0_0, H0_1, H1_0, H1_1, H2_0, H2_1):
    acc1 = _prop128_kernel(ys1.reshape(_NP * 8, 16), rc,
                           jnp.zeros((_ZRP, 16), jnp.float32))
    w4, zs = _tcc(Xp, acc0, dis, acc1, _kron4(H0_0), _kron4(H0_1),
                  _kron4(H1_0), _kron4(H1_1), _kron4(H2_0), _kron4(H2_1))
    return w4, zs


@jax.jit
def _stage4(rc, dis, w4, zs):
    acc2 = _prop16_kernel(zs, rc, jnp.zeros((_ZRP, 16), jnp.float32))
    out = _tcd(w4, dis, acc2)
    return out[:_N]


def kernel(X, edge_index, H0_0, H0_1, H1_0, H1_1, H2_0, H2_1):
    rc, Xp, dis, xs = _stage1(X, edge_index)
    acc0, ys1 = _stage2(rc, Xp, dis, xs, H0_0, H0_1)
    w4, zs = _stage3(rc, Xp, dis, acc0, ys1,
                     H0_0, H0_1, H1_0, H1_1, H2_0, H2_1)
    return _stage4(rc, dis, w4, zs)


# speedup vs baseline: 41.6134x; 1.0047x over previous
import functools

import jax
import jax.numpy as jnp
from jax import lax
from jax.experimental import pallas as pl
from jax.experimental.pallas import tpu as pltpu
from jax.experimental.pallas import tpu_sc as plsc

_N = 100000
_E = 1600000
_NP = 100352
_EP = 1605632
_NB = _EP // 128
_BT32 = _NB // 32
_BT16 = _NB // 16
_K = 8
_KP = 4
_NPT = _NP // 16
_ZRP = 196
_BN = 2048

_mesh = plsc.VectorSubcoreMesh(core_axis_name="c", subcore_axis_name="s")


def _elu(x):
    return jnp.where(x > 0.0, x, jnp.exp(jnp.minimum(x, 0.0)) - 1.0)


def _zero_acc(zbuf, acc, s, sem):
    descs = [
        pltpu.async_copy(zbuf, acc.at[pl.ds(s * _NPT + z * _ZRP, _ZRP)], sem)
        for z in range(_NPT // _ZRP)
    ]
    for d in descs:
        d.wait()


def _writeout(acc, wbuf, out, p, s, sem):
    del wbuf
    descs = [
        pltpu.async_copy(acc.at[pl.ds(s * _NPT + z * _ZRP, _ZRP)],
                         out.at[p, pl.ds(s * _NPT + z * _ZRP, _ZRP), :], sem)
        for z in range(_NPT // _ZRP)
    ]
    for d in descs:
        d.wait()


@functools.partial(
    pl.kernel,
    out_type=jax.ShapeDtypeStruct((2, _NP, 16), jnp.float32),
    mesh=_mesh,
    compiler_params=pltpu.CompilerParams(use_tc_tiling_on_sc=False),
    scratch_types=[
        pltpu.VMEM((_K, 2, 128), jnp.int32),
        pltpu.VMEM((128, 16), jnp.float32),
        pltpu.VMEM((_ZRP, 16), jnp.float32),
        pltpu.VMEM((_ZRP, 16), jnp.float32),
        pltpu.VMEM_SHARED((_NP, 16), jnp.float32),
        pltpu.SemaphoreType.DMA,
    ],
)
def _deg_kernel(rc, ones_hbm, zeros_hbm, out, sidx, ones_v, zbuf, wbuf,
                acc, sem):
    c = lax.axis_index("c")
    s = lax.axis_index("s")
    w = c * 16 + s
    pltpu.sync_copy(ones_hbm, ones_v)
    pltpu.sync_copy(zeros_hbm, zbuf)
    _zero_acc(zbuf, acc, s, sem)
    plsc.subcore_barrier()

    def body(i, carry):
        base = w * _BT32 + i * _K
        pltpu.sync_copy(rc.at[pl.ds(base, _K)], sidx)
        descs = [
            pltpu.async_copy(ones_v, acc.at[sidx.at[j, 1]], sem, add=True)
            for j in range(_K)
        ]
        for d in descs:
            d.wait()
        return carry

    lax.fori_loop(0, _BT32 // _K, body, 0)
    plsc.subcore_barrier()
    _writeout(acc, wbuf, out, c, s, sem)


@functools.partial(
    pl.kernel,
    out_type=jax.ShapeDtypeStruct((2, _NP, 16), jnp.float32),
    mesh=_mesh,
    compiler_params=pltpu.CompilerParams(use_tc_tiling_on_sc=False),
    scratch_types=[
        pltpu.VMEM((_KP, 2, 128), jnp.int32),
        pltpu.VMEM((_KP, 2, 128), jnp.int32),
        pltpu.VMEM((_KP * 128, 16), jnp.float32),
        pltpu.VMEM((_KP * 128, 16), jnp.float32),
        pltpu.VMEM((_ZRP, 16), jnp.float32),
        pltpu.VMEM((_ZRP, 16), jnp.float32),
        pltpu.VMEM_SHARED((_NP, 16), jnp.float32),
        pltpu.SemaphoreType.DMA,
        pltpu.SemaphoreType.DMA,
        pltpu.SemaphoreType.DMA,
    ],
)
def _prop16_kernel(tbl, rc, zeros_hbm, out,
                   rcA, rcB, rowsA, rowsB, zbuf, wbuf, acc,
                   semg, sems, semsb):
    c = lax.axis_index("c")
    s = lax.axis_index("s")
    w = c * 16 + s
    pltpu.sync_copy(zeros_hbm, zbuf)
    _zero_acc(zbuf, acc, s, semg)
    plsc.subcore_barrier()

    def _fire_gathers(rcb, rows, base):
        pltpu.sync_copy(rc.at[pl.ds(base, _KP)], rcb)
        return [
            pltpu.async_copy(tbl.at[rcb.at[j, 0]],
                             rows.at[pl.ds(j * 128, 128)], semg)
            for j in range(_KP)
        ]

    def _drain_b():
        for j in range(_KP):
            pltpu.make_async_copy(rowsB.at[pl.ds(j * 128, 128)],
                                  acc.at[rcB.at[j, 1]], semsb).wait()

    def body(i, carry):
        base = w * _BT32 + i * (2 * _KP)
        gA = _fire_gathers(rcA, rowsA, base)

        @pl.when(i > 0)
        def _():
            _drain_b()

        gB = _fire_gathers(rcB, rowsB, base + _KP)
        sds = []
        for j in range(_KP):
            gA[j].wait()
            sds.append(pltpu.async_copy(
                rowsA.at[pl.ds(j * 128, 128)],
                acc.at[rcA.at[j, 1]], sems, add=True))
        for j in range(_KP):
            gB[j].wait()
            pltpu.async_copy(rowsB.at[pl.ds(j * 128, 128)],
                             acc.at[rcB.at[j, 1]], semsb, add=True)
        for d in sds:
            d.wait()
        return carry

    lax.fori_loop(0, _BT32 // (2 * _KP), body, 0)
    _drain_b()
    plsc.subcore_barrier()
    _writeout(acc, wbuf, out, c, s, semg)


@functools.partial(
    pl.kernel,
    out_type=jax.ShapeDtypeStruct((8, _NP, 16), jnp.float32),
    mesh=_mesh,
    compiler_params=pltpu.CompilerParams(use_tc_tiling_on_sc=False),
    scratch_types=[
        pltpu.VMEM((_KP, 2, 128), jnp.int32),
        pltpu.VMEM((_KP, 2, 128), jnp.int32),
        pltpu.VMEM((_KP * 128, 16), jnp.float32),
        pltpu.VMEM((_KP * 128, 16), jnp.float32),
        pltpu.VMEM((_ZRP, 16), jnp.float32),
        pltpu.VMEM((_ZRP, 16), jnp.float32),
        pltpu.VMEM_SHARED((_NP, 16), jnp.float32),
        pltpu.SemaphoreType.DMA,
        pltpu.SemaphoreType.DMA,
        pltpu.SemaphoreType.DMA,
    ],
)
def _prop128_kernel(tbl, rc, zeros_hbm, out,
                    rcA, rcB, rowsA, rowsB, zbuf, wbuf, acc,
                    semg, sems, semsb):
    c = lax.axis_index("c")
    s = lax.axis_index("s")
    pltpu.sync_copy(zeros_hbm, zbuf)
    for pstep in range(4):
        p = pstep * 2 + c
        _zero_acc(zbuf, acc, s, semg)
        plsc.subcore_barrier()

        def _fire_gathers(rcb, rows, base):
            pltpu.sync_copy(rc.at[pl.ds(base, _KP)], rcb)
            for j in range(_KP):
                for l in range(8):
                    v = rcb[j, 0, pl.ds(l * 16, 16)]
                    rcb[j, 0, pl.ds(l * 16, 16)] = v * 8 + p
            return [
                pltpu.async_copy(tbl.at[rcb.at[j, 0]],
                                 rows.at[pl.ds(j * 128, 128)], semg)
                for j in range(_KP)
            ]

        def _drain_b():
            for j in range(_KP):
                pltpu.make_async_copy(rowsB.at[pl.ds(j * 128, 128)],
                                      acc.at[rcB.at[j, 1]], semsb).wait()

        def body(i, carry):
            base = s * _BT16 + i * (2 * _KP)
            gA = _fire_gathers(rcA, rowsA, base)

            @pl.when(i > 0)
            def _():
                _drain_b()

            gB = _fire_gathers(rcB, rowsB, base + _KP)
            sds = []
            for j in range(_KP):
                gA[j].wait()
                sds.append(pltpu.async_copy(
                    rowsA.at[pl.ds(j * 128, 128)],
                    acc.at[rcA.at[j, 1]], sems, add=True))
            for j in range(_KP):
                gB[j].wait()
                pltpu.async_copy(rowsB.at[pl.ds(j * 128, 128)],
                                 acc.at[rcB.at[j, 1]], semsb, add=True)
            for d in sds:
                d.wait()
            return carry

        lax.fori_loop(0, _BT16 // (2 * _KP), body, 0)
        _drain_b()
        plsc.subcore_barrier()
        _writeout(acc, wbuf, out, p, s, semg)
        plsc.subcore_barrier()


def _tca_body(deg_ref, x_ref, dis_ref, xs_ref):
    deg = deg_ref[0, :, 0:1] + deg_ref[1, :, 0:1]
    dis = jnp.where(deg > 0.0, lax.rsqrt(jnp.maximum(deg, 1e-12)), 0.0)
    dis_ref[...] = dis
    xs = dis * x_ref[...]
    xs_ref[...] = jnp.concatenate(
        [xs, jnp.zeros((xs.shape[0], 12), jnp.float32)], axis=1)


_tca = pl.pallas_call(
    _tca_body,
    grid=(_NP // _BN,),
    in_specs=[
        pl.BlockSpec((2, _BN, 16), lambda i: (0, i, 0)),
        pl.BlockSpec((_BN, 4), lambda i: (i, 0)),
    ],
    out_specs=[
        pl.BlockSpec((_BN, 1), lambda i: (i, 0)),
        pl.BlockSpec((_BN, 16), lambda i: (i, 0)),
    ],
    out_shape=[
        jax.ShapeDtypeStruct((_NP, 1), jnp.float32),
        jax.ShapeDtypeStruct((_NP, 16), jnp.float32),
    ],
)


def _tcb_body(x_ref, a0_ref, dis_ref, h00k_ref, h01k_ref, ys_ref):
    dis = dis_ref[...]
    p0 = dis * (a0_ref[0, :, 0:4] + a0_ref[1, :, 0:4])
    x = x_ref[...]
    y1 = _elu(jnp.dot(x, h00k_ref[...], preferred_element_type=jnp.float32)
              + jnp.dot(p0, h01k_ref[...], preferred_element_type=jnp.float32))
    ys_ref[...] = dis * y1


_tcb = pl.pallas_call(
    _tcb_body,
    grid=(_NP // _BN,),
    in_specs=[
        pl.BlockSpec((_BN, 4), lambda i: (i, 0)),
        pl.BlockSpec((2, _BN, 16), lambda i: (0, i, 0)),
        pl.BlockSpec((_BN, 1), lambda i: (i, 0)),
        pl.BlockSpec((4, 128), lambda i: (0, 0)),
        pl.BlockSpec((4, 128), lambda i: (0, 0)),
    ],
    out_specs=pl.BlockSpec((_BN, 128), lambda i: (i, 0)),
    out_shape=jax.ShapeDtypeStruct((_NP, 128), jnp.float32),
)


def _tcc_body(x_ref, a0_ref, dis_ref, a1_ref,
              h00k_ref, h01k_ref, h10k_ref, h11k_ref, h20k_ref, h21k_ref,
              w_ref, zs_ref):
    dis = dis_ref[...]
    p0 = dis * (a0_ref[0, :, 0:4] + a0_ref[1, :, 0:4])
    x = x_ref[...]
    a1 = a1_ref[...]
    y1 = _elu(jnp.dot(x, h00k_ref[...], preferred_element_type=jnp.float32)
              + jnp.dot(p0, h01k_ref[...], preferred_element_type=jnp.float32))
    p1 = dis * jnp.concatenate([a1[q] for q in range(8)], axis=1)
    y2 = _elu(
        jnp.dot(y1, h10k_ref[...], preferred_element_type=jnp.float32)
        + jnp.dot(p1, h11k_ref[...], preferred_element_type=jnp.float32))
    w_ref[...] = jnp.dot(y2, h20k_ref[...],
                         preferred_element_type=jnp.float32)
    zs = dis * jnp.dot(y2, h21k_ref[...], preferred_element_type=jnp.float32)
    zs_ref[...] = jnp.concatenate(
        [zs, jnp.zeros((zs.shape[0], 12), jnp.float32)], axis=1)


_tcc = pl.pallas_call(
    _tcc_body,
    grid=(_NP // _BN,),
    compiler_params=pltpu.CompilerParams(vmem_limit_bytes=100 * 1024 * 1024),
    in_specs=[
        pl.BlockSpec((_BN, 4), lambda i: (i, 0)),
        pl.BlockSpec((2, _BN, 16), lambda i: (0, i, 0)),
        pl.BlockSpec((_BN, 1), lambda i: (i, 0)),
        pl.BlockSpec((8, _BN, 16), lambda i: (0, i, 0)),
        pl.BlockSpec((4, 128), lambda i: (0, 0)),
        pl.BlockSpec((4, 128), lambda i: (0, 0)),
        pl.BlockSpec((128, 128), lambda i: (0, 0)),
        pl.BlockSpec((128, 128), lambda i: (0, 0)),
        pl.BlockSpec((128, 4), lambda i: (0, 0)),
        pl.BlockSpec((128, 4), lambda i: (0, 0)),
    ],
    out_specs=[
        pl.BlockSpec((_BN, 4), lambda i: (i, 0)),
        pl.BlockSpec((_BN, 16), lambda i: (i, 0)),
    ],
    out_shape=[
        jax.ShapeDtypeStruct((_NP, 4), jnp.float32),
        jax.ShapeDtypeStruct((_NP, 16), jnp.float32),
    ],
)


def _tcd_body(w_ref, dis_ref, a2_ref, o_ref):
    o_ref[...] = w_ref[...] + dis_ref[...] * (
        a2_ref[0, :, 0:4] + a2_ref[1, :, 0:4])


_tcd = pl.pallas_call(
    _tcd_body,
    grid=(_NP // _BN,),
    in_specs=[
        pl.BlockSpec((_BN, 4), lambda i: (i, 0)),
        pl.BlockSpec((_BN, 1), lambda i: (i, 0)),
        pl.BlockSpec((2, _BN, 16), lambda i: (0, i, 0)),
    ],
    out_specs=pl.BlockSpec((_BN, 4), lambda i: (i, 0)),
    out_shape=jax.ShapeDtypeStruct((_NP, 4), jnp.float32),
)


@jax.jit
def _stage1(X, edge_index):
    row = edge_index[0].astype(jnp.int32)
    col = edge_index[1].astype(jnp.int32)
    rowp = jnp.concatenate(
        [row, jnp.zeros((_EP - _E,), jnp.int32)]).reshape(_NB, 128)
    colp = jnp.concatenate(
        [col, jnp.full((_EP - _E,), _N, jnp.int32)]).reshape(_NB, 128)
    rc = jnp.stack([rowp, colp], axis=1)
    Xp = jnp.pad(X, ((0, _NP - _N), (0, 0)))
    deg2 = _deg_kernel(rc, jnp.ones((128, 16), jnp.float32),
                       jnp.zeros((_ZRP, 16), jnp.float32))
    dis, xs = _tca(deg2, Xp)
    return rc, Xp, dis, xs


def _kron4(h):
    return jnp.kron(jnp.eye(4, dtype=jnp.float32), h)


@jax.jit
def _stage2(rc, Xp, dis, xs, H0_0, H0_1):
    acc0 = _prop16_kernel(xs, rc, jnp.zeros((_ZRP, 16), jnp.float32))
    ys1 = _tcb(Xp, acc0, dis, _kron4(H0_0), _kron4(H0_1))
    return acc0, ys1


@jax.jit
def _stage3(rc, Xp, dis, acc0, ys1,
            H0_0, H0_1, H1_0, H1_1, H2_0, H2_1):
    acc1 = _prop128_kernel(ys1.reshape(_NP * 8, 16), rc,
                           jnp.zeros((_ZRP, 16), jnp.float32))
    w4, zs = _tcc(Xp, acc0, dis, acc1, _kron4(H0_0), _kron4(H0_1),
                  _kron4(H1_0), _kron4(H1_1), _kron4(H2_0), _kron4(H2_1))
    return w4, zs


@jax.jit
def _stage4(rc, dis, w4, zs):
    acc2 = _prop16_kernel(zs, rc, jnp.zeros((_ZRP, 16), jnp.float32))
    out = _tcd(w4, dis, acc2)
    return out[:_N]


def kernel(X, edge_index, H0_0, H0_1, H1_0, H1_1, H2_0, H2_1):
    rc, Xp, dis, xs = _stage1(X, edge_index)
    acc0, ys1 = _stage2(rc, Xp, dis, xs, H0_0, H0_1)
    w4, zs = _stage3(rc, Xp, dis, acc0, ys1,
                     H0_0, H0_1, H1_0, H1_1, H2_0, H2_1)
    return _stage4(rc, dis, w4, zs)
